# baseline (speedup 1.0000x reference)
import functools

import jax
import jax.numpy as jnp
from jax import lax
from jax.experimental import pallas as pl
from jax.experimental.pallas import tpu as pltpu
from jax.experimental.pallas import tpu_sc as plsc

MAX_REL_K = 512
HEADS = 16
SEQ = 2048
TBL = 2 * MAX_REL_K + 1
WROW = 1032
PV_LEN = 4096
LANES = 16
NCORES = 2
NBAND = 10
NTILE = SEQ // 128


def _make_sc_kernel():
    mesh = plsc.VectorSubcoreMesh(core_axis_name="c", subcore_axis_name="s")

    @functools.partial(
        pl.kernel,
        mesh=mesh,
        compiler_params=pltpu.CompilerParams(needs_layout_passes=False),
        out_type=jax.ShapeDtypeStruct((HEADS, SEQ, SEQ), jnp.float32),
        scratch_types=[
            pltpu.VMEM((WROW,), jnp.float32),
        ]
        + [pltpu.VMEM((PV_LEN,), jnp.float32) for _ in range(16)]
        + [pltpu.VMEM((8, SEQ), jnp.float32) for _ in range(2)]
        + [pltpu.VMEM((8, 128), jnp.float32) for _ in range(2)]
        + [pltpu.SemaphoreType.DMA for _ in range(2)],
    )
    def body(wt_hbm, out_hbm, wh_v, *rest):
        pv_v = rest[:16]
        stg = rest[16:18]
        cst = rest[18:20]
        sems = rest[20:22]
        wid = lax.axis_index("s") * NCORES + lax.axis_index("c")
        h = wid // 2
        g0 = (wid % 2) * (SEQ // 2 // 8)

        pltpu.sync_copy(wt_hbm.at[h], wh_v)

        lane = lax.iota(jnp.int32, LANES)
        for r in range(16):
            def build(c, _, r=r):
                u = c * LANES + lane + (r - (SEQ - 1))
                idx = jnp.clip(u, -MAX_REL_K, MAX_REL_K) + MAX_REL_K
                pv_v[r][pl.ds(c * LANES, LANES)] = plsc.load_gather(wh_v, [idx])
                return 0

            lax.fori_loop(0, PV_LEN // LANES, build, 0)

        w_lo = pv_v[0][pl.ds(0, LANES)]
        w_hi = pv_v[0][pl.ds(PV_LEN - LANES, LANES)]
        for p in range(8):
            for t in range(8):
                cst[0][p, pl.ds(16 * t, LANES)] = w_lo
                cst[1][p, pl.ds(16 * t, LANES)] = w_hi

        def group(g, b, r0, first):
            m16 = (SEQ - 1) - r0 - 8 * g
            cms = jnp.clip((8 * g - 512) // 128, 0, NTILE - NBAND)
            if not first:
                for c in range(NTILE):
                    pltpu.make_async_copy(
                        stg[b].at[:, pl.ds(128 * c, 128)],
                        out_hbm.at[h, pl.ds(8 * g, 8), pl.ds(128 * c, 128)],
                        sems[b],
                    ).wait()
            for c in range(NTILE):
                in_band = jnp.logical_and(c >= cms, c < cms + NBAND)
                dst = out_hbm.at[h, pl.ds(8 * g, 8), pl.ds(128 * c, 128)]

                @pl.when(in_band)
                def _(c=c):
                    @plsc.parallel_loop(0, 8, unroll=8)
                    def ft(t):
                        off = 128 * c + 16 * t
                        so = m16 + off
                        for p in range(8):
                            stg[b][p, pl.ds(off, LANES)] = pv_v[r0 - p][
                                pl.ds(so, LANES)
                            ]

                    pltpu.async_copy(
                        stg[b].at[:, pl.ds(128 * c, 128)], dst, sems[b]
                    )

                @pl.when(c < cms)
                def _():
                    pltpu.async_copy(cst[0], dst, sems[b])

                @pl.when(c >= cms + NBAND)
                def _():
                    pltpu.async_copy(cst[1], dst, sems[b])

        group(g0, 0, 15, True)
        group(g0 + 1, 1, 7, True)

        def pipe_steady(u, _):
            g = g0 + 2 * u
            group(g, 0, 15, False)
            group(g + 1, 1, 7, False)
            return 0

        lax.fori_loop(1, SEQ // 2 // 8 // 2, pipe_steady, 0)

        gl = g0 + SEQ // 2 // 8 - 2
        for b, g in ((0, gl), (1, gl + 1)):
            for c in range(NTILE):
                pltpu.make_async_copy(
                    stg[b].at[:, pl.ds(128 * c, 128)],
                    out_hbm.at[h, pl.ds(8 * g, 8), pl.ds(128 * c, 128)],
                    sems[b],
                ).wait()

    return body


_sc_bias = _make_sc_kernel()


def kernel(qlen, klen, W):
    wt = jnp.zeros((HEADS, WROW), jnp.float32).at[:, :TBL].set(W.T)
    return _sc_bias(wt)


# speedup vs baseline: 1.4288x; 1.4288x over previous
import functools

import jax
import jax.numpy as jnp
from jax import lax
from jax.experimental import pallas as pl
from jax.experimental.pallas import tpu as pltpu
from jax.experimental.pallas import tpu_sc as plsc

MAX_REL_K = 512
HEADS = 16
SEQ = 2048
TBL = 2 * MAX_REL_K + 1
WROW = 1032
PV_LEN = 4096
LANES = 16
NCORES = 2
NBAND = 10
NTILE = SEQ // 128


def _make_sc_kernel():
    mesh = plsc.VectorSubcoreMesh(core_axis_name="c", subcore_axis_name="s")

    @functools.partial(
        pl.kernel,
        mesh=mesh,
        compiler_params=pltpu.CompilerParams(needs_layout_passes=False),
        out_type=jax.ShapeDtypeStruct((HEADS, SEQ, SEQ), jnp.float32),
        scratch_types=[
            pltpu.VMEM((WROW,), jnp.float32),
        ]
        + [pltpu.VMEM((PV_LEN,), jnp.float32) for _ in range(16)]
        + [pltpu.VMEM((8, SEQ), jnp.float32) for _ in range(2)]
        + [pltpu.VMEM((8, 128), jnp.float32) for _ in range(2)]
        + [pltpu.SemaphoreType.DMA for _ in range(2)],
    )
    def body(wt_hbm, out_hbm, wh_v, *rest):
        pv_v = rest[:16]
        stg = rest[16:18]
        cst = rest[18:20]
        sems = rest[20:22]
        wid = lax.axis_index("s") * NCORES + lax.axis_index("c")
        h = wid // 2
        g0 = (wid % 2) * (SEQ // 2 // 8)

        pltpu.sync_copy(wt_hbm.at[h], wh_v)

        lane = lax.iota(jnp.int32, LANES)
        for r in range(16):
            def build(c, _, r=r):
                u = c * LANES + lane + (r - (SEQ - 1))
                idx = jnp.clip(u, -MAX_REL_K, MAX_REL_K) + MAX_REL_K
                pv_v[r][pl.ds(c * LANES, LANES)] = plsc.load_gather(wh_v, [idx])
                return 0

            lax.fori_loop(0, PV_LEN // LANES, build, 0)

        w_lo = pv_v[0][pl.ds(0, LANES)]
        w_hi = pv_v[0][pl.ds(PV_LEN - LANES, LANES)]
        for p in range(8):
            for t in range(8):
                cst[0][p, pl.ds(16 * t, LANES)] = w_lo
                cst[1][p, pl.ds(16 * t, LANES)] = w_hi

        def group(g, b, r0, first):
            m16 = (SEQ - 1) - r0 - 8 * g
            cms = jnp.clip((8 * g - 512) // 128, 0, NTILE - NBAND)
            if not first:
                for c in range(NTILE):
                    pltpu.make_async_copy(
                        stg[b].at[:, pl.ds(128 * c, 128)],
                        out_hbm.at[h, pl.ds(8 * g, 8), pl.ds(128 * c, 128)],
                        sems[b],
                    ).wait()
            for c in range(NTILE):
                in_band = jnp.logical_and(c >= cms, c < cms + NBAND)
                dst = out_hbm.at[h, pl.ds(8 * g, 8), pl.ds(128 * c, 128)]

                @pl.when(in_band)
                def _(c=c):
                    @plsc.parallel_loop(0, 8, unroll=4)
                    def ft(t):
                        off = 128 * c + 16 * t
                        so = m16 + off
                        for p in range(8):
                            stg[b][p, pl.ds(off, LANES)] = pv_v[r0 - p][
                                pl.ds(so, LANES)
                            ]

                    pltpu.async_copy(
                        stg[b].at[:, pl.ds(128 * c, 128)], dst, sems[b]
                    )

                @pl.when(c < cms)
                def _():
                    pltpu.async_copy(cst[0], dst, sems[b])

                @pl.when(c >= cms + NBAND)
                def _():
                    pltpu.async_copy(cst[1], dst, sems[b])

        group(g0, 0, 15, True)
        group(g0 + 1, 1, 7, True)

        def pipe_steady(u, _):
            g = g0 + 2 * u
            group(g, 0, 15, False)
            group(g + 1, 1, 7, False)
            return 0

        lax.fori_loop(1, SEQ // 2 // 8 // 2, pipe_steady, 0)

        gl = g0 + SEQ // 2 // 8 - 2
        for b, g in ((0, gl), (1, gl + 1)):
            for c in range(NTILE):
                pltpu.make_async_copy(
                    stg[b].at[:, pl.ds(128 * c, 128)],
                    out_hbm.at[h, pl.ds(8 * g, 8), pl.ds(128 * c, 128)],
                    sems[b],
                ).wait()

    return body


_sc_bias = _make_sc_kernel()


def kernel(qlen, klen, W):
    wt = jnp.zeros((HEADS, WROW), jnp.float32).at[:, :TBL].set(W.T)
    return _sc_bias(wt)


# speedup vs baseline: 2.1660x; 1.5160x over previous
import functools

import jax
import jax.numpy as jnp
from jax import lax
from jax.experimental import pallas as pl
from jax.experimental.pallas import tpu as pltpu
from jax.experimental.pallas import tpu_sc as plsc

MAX_REL_K = 512
HEADS = 16
SEQ = 2048
TBL = 2 * MAX_REL_K + 1
WROW = 1032
PV_LEN = 4096
LANES = 16
NCORES = 2
NBAND = 11
NTILE = SEQ // 128


def _make_sc_kernel():
    mesh = plsc.VectorSubcoreMesh(core_axis_name="c", subcore_axis_name="s")

    @functools.partial(
        pl.kernel,
        mesh=mesh,
        compiler_params=pltpu.CompilerParams(needs_layout_passes=False),
        out_type=jax.ShapeDtypeStruct((HEADS, SEQ, SEQ), jnp.float32),
        scratch_types=[
            pltpu.VMEM((WROW,), jnp.float32),
        ]
        + [pltpu.VMEM((PV_LEN,), jnp.float32) for _ in range(16)]
        + [pltpu.VMEM((8, SEQ), jnp.float32) for _ in range(2)]
        + [pltpu.VMEM((8, 128), jnp.float32) for _ in range(2)]
        + [pltpu.SemaphoreType.DMA for _ in range(3)],
    )
    def body(wt_hbm, out_hbm, wh_v, *rest):
        pv_v = rest[:16]
        stg = rest[16:18]
        cst = rest[18:20]
        sems = rest[20:22]
        sem_c = rest[22]
        wid = lax.axis_index("s") * NCORES + lax.axis_index("c")
        h = wid // 2
        g0 = (wid % 2) * (SEQ // 2 // 8)

        pltpu.sync_copy(wt_hbm.at[h], wh_v)

        lane = lax.iota(jnp.int32, LANES)
        for r in range(16):
            def build(c, _, r=r):
                u = c * LANES + lane + (r - (SEQ - 1))
                idx = jnp.clip(u, -MAX_REL_K, MAX_REL_K) + MAX_REL_K
                pv_v[r][pl.ds(c * LANES, LANES)] = plsc.load_gather(wh_v, [idx])
                return 0

            lax.fori_loop(0, PV_LEN // LANES, build, 0)

        w_lo = pv_v[0][pl.ds(0, LANES)]
        w_hi = pv_v[0][pl.ds(PV_LEN - LANES, LANES)]
        for p in range(8):
            for t in range(8):
                cst[0][p, pl.ds(16 * t, LANES)] = w_lo
                cst[1][p, pl.ds(16 * t, LANES)] = w_hi

        def group(g, b, r0, first):
            m16 = (SEQ - 1) - r0 - 8 * g
            cms = jnp.clip((8 * g - 639) // 128, 0, NTILE - NBAND)
            base = pl.multiple_of(128 * cms, 128)

            def lfire(c, _):
                pltpu.async_copy(
                    cst[0],
                    out_hbm.at[
                        h,
                        pl.ds(8 * g, 8),
                        pl.ds(pl.multiple_of(128 * c, 128), 128),
                    ],
                    sem_c,
                )
                return 0

            def rfire(c, _):
                cc = cms + NBAND + c
                pltpu.async_copy(
                    cst[1],
                    out_hbm.at[
                        h,
                        pl.ds(8 * g, 8),
                        pl.ds(pl.multiple_of(128 * cc, 128), 128),
                    ],
                    sem_c,
                )
                return 0

            lax.fori_loop(0, cms, lfire, 0)
            lax.fori_loop(0, (NTILE - NBAND) - cms, rfire, 0)

            if not first:
                pltpu.make_async_copy(
                    stg[b].at[:, pl.ds(base, 128 * NBAND)],
                    out_hbm.at[h, pl.ds(8 * g, 8), pl.ds(base, 128 * NBAND)],
                    sems[b],
                ).wait()

            @plsc.parallel_loop(0, 8 * NBAND, unroll=4)
            def ft(t):
                off = base + 16 * t
                so = m16 + off
                for p in range(8):
                    stg[b][p, pl.ds(off, LANES)] = pv_v[r0 - p][
                        pl.ds(so, LANES)
                    ]

            pltpu.async_copy(
                stg[b].at[:, pl.ds(base, 128 * NBAND)],
                out_hbm.at[h, pl.ds(8 * g, 8), pl.ds(base, 128 * NBAND)],
                sems[b],
            )

            for _ in range(NTILE - NBAND):
                pltpu.make_async_copy(
                    cst[0],
                    out_hbm.at[h, pl.ds(8 * g, 8), pl.ds(0, 128)],
                    sem_c,
                ).wait()

        group(g0, 0, 15, True)
        group(g0 + 1, 1, 7, True)

        def pipe_steady(u, _):
            g = g0 + 2 * u
            group(g, 0, 15, False)
            group(g + 1, 1, 7, False)
            return 0

        lax.fori_loop(1, SEQ // 2 // 8 // 2, pipe_steady, 0)

        gl = g0 + SEQ // 2 // 8 - 2
        for b, g in ((0, gl), (1, gl + 1)):
            cms = jnp.clip((8 * g - 639) // 128, 0, NTILE - NBAND)
            base = pl.multiple_of(128 * cms, 128)
            pltpu.make_async_copy(
                stg[b].at[:, pl.ds(base, 128 * NBAND)],
                out_hbm.at[h, pl.ds(8 * g, 8), pl.ds(base, 128 * NBAND)],
                sems[b],
            ).wait()

    return body


_sc_bias = _make_sc_kernel()


def kernel(qlen, klen, W):
    wt = jnp.zeros((HEADS, WROW), jnp.float32).at[:, :TBL].set(W.T)
    return _sc_bias(wt)


# speedup vs baseline: 2.1784x; 1.0057x over previous
import functools

import jax
import jax.numpy as jnp
from jax import lax
from jax.experimental import pallas as pl
from jax.experimental.pallas import tpu as pltpu
from jax.experimental.pallas import tpu_sc as plsc

MAX_REL_K = 512
HEADS = 16
SEQ = 2048
TBL = 2 * MAX_REL_K + 1
WROW = 1032
PV_LEN = 4096
LANES = 16
NCORES = 2
NBAND = 11
NTILE = SEQ // 128


def _make_sc_kernel():
    mesh = plsc.VectorSubcoreMesh(core_axis_name="c", subcore_axis_name="s")

    @functools.partial(
        pl.kernel,
        mesh=mesh,
        compiler_params=pltpu.CompilerParams(needs_layout_passes=False),
        out_type=jax.ShapeDtypeStruct((HEADS, SEQ, SEQ), jnp.float32),
        scratch_types=[
            pltpu.VMEM((WROW,), jnp.float32),
        ]
        + [pltpu.VMEM((PV_LEN,), jnp.float32) for _ in range(16)]
        + [pltpu.VMEM((8, SEQ), jnp.float32) for _ in range(2)]
        + [pltpu.VMEM((8, 128), jnp.float32) for _ in range(2)]
        + [pltpu.SemaphoreType.DMA for _ in range(3)],
    )
    def body(wt_hbm, out_hbm, wh_v, *rest):
        pv_v = rest[:16]
        stg = rest[16:18]
        cst = rest[18:20]
        sems = rest[20:22]
        sem_c = rest[22]
        wid = lax.axis_index("s") * NCORES + lax.axis_index("c")
        h = wid // 2
        g0 = (wid % 2) * (SEQ // 2 // 8)

        pltpu.sync_copy(wt_hbm.at[h], wh_v)

        lane = lax.iota(jnp.int32, LANES)
        for r in range(16):
            def build(c, _, r=r):
                u = c * LANES + lane + (r - (SEQ - 1))
                idx = jnp.clip(u, -MAX_REL_K, MAX_REL_K) + MAX_REL_K
                pv_v[r][pl.ds(c * LANES, LANES)] = plsc.load_gather(wh_v, [idx])
                return 0

            lax.fori_loop(0, PV_LEN // LANES, build, 0)

        w_lo = pv_v[0][pl.ds(0, LANES)]
        w_hi = pv_v[0][pl.ds(PV_LEN - LANES, LANES)]
        for p in range(8):
            for t in range(8):
                cst[0][p, pl.ds(16 * t, LANES)] = w_lo
                cst[1][p, pl.ds(16 * t, LANES)] = w_hi

        def group(g, b, r0, first):
            m16 = (SEQ - 1) - r0 - 8 * g
            cms = jnp.clip((8 * g - 639) // 128, 0, NTILE - NBAND)
            base = pl.multiple_of(128 * cms, 128)

            def lfire(c, _):
                pltpu.async_copy(
                    cst[0],
                    out_hbm.at[
                        h,
                        pl.ds(8 * g, 8),
                        pl.ds(pl.multiple_of(128 * c, 128), 128),
                    ],
                    sem_c,
                )
                return 0

            def rfire(c, _):
                cc = cms + NBAND + c
                pltpu.async_copy(
                    cst[1],
                    out_hbm.at[
                        h,
                        pl.ds(8 * g, 8),
                        pl.ds(pl.multiple_of(128 * cc, 128), 128),
                    ],
                    sem_c,
                )
                return 0

            lax.fori_loop(0, cms, lfire, 0)
            lax.fori_loop(0, (NTILE - NBAND) - cms, rfire, 0)

            if not first:
                pltpu.make_async_copy(
                    stg[b].at[:, pl.ds(base, 128 * NBAND)],
                    out_hbm.at[h, pl.ds(8 * g, 8), pl.ds(base, 128 * NBAND)],
                    sems[b],
                ).wait()

            @plsc.parallel_loop(0, 8 * NBAND, unroll=8)
            def ft(t):
                off = base + 16 * t
                so = m16 + off
                for p in range(8):
                    stg[b][p, pl.ds(off, LANES)] = pv_v[r0 - p][
                        pl.ds(so, LANES)
                    ]

            pltpu.async_copy(
                stg[b].at[:, pl.ds(base, 128 * NBAND)],
                out_hbm.at[h, pl.ds(8 * g, 8), pl.ds(base, 128 * NBAND)],
                sems[b],
            )

            for _ in range(NTILE - NBAND):
                pltpu.make_async_copy(
                    cst[0],
                    out_hbm.at[h, pl.ds(8 * g, 8), pl.ds(0, 128)],
                    sem_c,
                ).wait()

        group(g0, 0, 15, True)
        group(g0 + 1, 1, 7, True)

        def pipe_steady(u, _):
            g = g0 + 2 * u
            group(g, 0, 15, False)
            group(g + 1, 1, 7, False)
            return 0

        lax.fori_loop(1, SEQ // 2 // 8 // 2, pipe_steady, 0)

        gl = g0 + SEQ // 2 // 8 - 2
        for b, g in ((0, gl), (1, gl + 1)):
            cms = jnp.clip((8 * g - 639) // 128, 0, NTILE - NBAND)
            base = pl.multiple_of(128 * cms, 128)
            pltpu.make_async_copy(
                stg[b].at[:, pl.ds(base, 128 * NBAND)],
                out_hbm.at[h, pl.ds(8 * g, 8), pl.ds(base, 128 * NBAND)],
                sems[b],
            ).wait()

    return body


_sc_bias = _make_sc_kernel()


def kernel(qlen, klen, W):
    wt = jnp.zeros((HEADS, WROW), jnp.float32).at[:, :TBL].set(W.T)
    return _sc_bias(wt)
